# Initial kernel scaffold; baseline (speedup 1.0000x reference)
import jax
import jax.numpy as jnp
from jax.experimental import pallas as pl


def kernel(input_word_ids, input_type_ids, word_emb, pos_emb, type_emb, ln_gamma, ln_beta):
    raise NotImplementedError("write your pallas kernel here")


# speedup vs baseline: 1.7206x; 1.7206x over previous
import functools

import jax
import jax.numpy as jnp
from jax import lax
from jax.experimental import pallas as pl
from jax.experimental.pallas import tpu as pltpu
from jax.experimental.pallas import tpu_sc as plsc

BATCH = 4
SEQ = 2048
TOKENS = BATCH * SEQ
D = 1024
LN_EPS = 1e-3

NW = 32
TPW = TOKENS // NW
CHUNK = 32
N_CHUNKS = TPW // CHUNK

_sc_mesh = plsc.VectorSubcoreMesh(core_axis_name="c", subcore_axis_name="s")


@functools.partial(
    pl.kernel,
    mesh=_sc_mesh,
    out_type=jax.ShapeDtypeStruct((TOKENS, D), jnp.float32),
    scratch_types=[
        pltpu.VMEM((TPW,), jnp.int32),
        pltpu.VMEM((CHUNK, D), jnp.float32),
        pltpu.VMEM((CHUNK, D), jnp.float32),
        pltpu.SemaphoreType.DMA,
        pltpu.SemaphoreType.DMA,
    ],
)
def _sc_gather(idx_hbm, table_hbm, out_hbm, idx_v, rows_a, rows_b, sem_a, sem_b):
    wid = lax.axis_index("s") * 2 + lax.axis_index("c")
    base = wid * TPW
    pltpu.sync_copy(idx_hbm.at[pl.ds(base, TPW)], idx_v)

    pltpu.async_copy(table_hbm.at[idx_v.at[pl.ds(0, CHUNK)]], rows_a, sem_a)

    def body(i, _):
        buf = [rows_a, rows_b]
        sem = [sem_a, sem_b]
        for p in range(2):
            @pl.when(lax.rem(i, 2) == p)
            def _():
                cur, nxt = buf[p], buf[1 - p]
                csem, nsem = sem[p], sem[1 - p]
                @pl.when(i + 1 < N_CHUNKS)
                def _():
                    pltpu.async_copy(
                        table_hbm.at[idx_v.at[pl.ds((i + 1) * CHUNK, CHUNK)]],
                        nxt, nsem)
                pltpu.make_async_copy(
                    table_hbm.at[idx_v.at[pl.ds(i * CHUNK, CHUNK)]], cur, csem
                ).wait()
                pltpu.sync_copy(cur, out_hbm.at[pl.ds(base + i * CHUNK, CHUNK)])
        return 0

    lax.fori_loop(0, N_CHUNKS, body, 0)


TC_BLK = 512


def _tc_ln_body(gathered_ref, pos_ref, tsel_ref, type_emb_ref, gamma_ref,
                beta_ref, out_ref):
    x = gathered_ref[...]
    t0 = type_emb_ref[0:1, :]
    t1 = type_emb_ref[1:2, :]
    tsel = tsel_ref[...]
    x = x + pos_ref[...] + t0 + tsel * (t1 - t0)
    mean = jnp.mean(x, axis=-1, keepdims=True)
    xc = x - mean
    var = jnp.mean(xc * xc, axis=-1, keepdims=True)
    y = xc * lax.rsqrt(var + LN_EPS)
    out_ref[...] = y * gamma_ref[...] + beta_ref[...]


_BLOCKS_PER_BATCH = SEQ // TC_BLK


_tc_ln = pl.pallas_call(
    _tc_ln_body,
    grid=(TOKENS // TC_BLK,),
    in_specs=[
        pl.BlockSpec((TC_BLK, D), lambda i: (i, 0)),
        pl.BlockSpec((TC_BLK, D), lambda i: (i % _BLOCKS_PER_BATCH, 0)),
        pl.BlockSpec((TC_BLK, 1), lambda i: (i, 0)),
        pl.BlockSpec((2, D), lambda i: (0, 0)),
        pl.BlockSpec((1, D), lambda i: (0, 0)),
        pl.BlockSpec((1, D), lambda i: (0, 0)),
    ],
    out_specs=pl.BlockSpec((TC_BLK, D), lambda i: (i, 0)),
    out_shape=jax.ShapeDtypeStruct((TOKENS, D), jnp.float32),
)


def kernel(input_word_ids, input_type_ids, word_emb, pos_emb, type_emb,
           ln_gamma, ln_beta):
    ids32 = input_word_ids.reshape(TOKENS).astype(jnp.int32)
    gathered = _sc_gather(ids32, word_emb)
    tsel = input_type_ids.reshape(TOKENS, 1).astype(jnp.float32)
    out = _tc_ln(gathered, pos_emb, tsel, type_emb,
                 ln_gamma.reshape(1, D), ln_beta.reshape(1, D))
    return out.reshape(BATCH, SEQ, D)


# speedup vs baseline: 1.7910x; 1.0409x over previous
import functools

import jax
import jax.numpy as jnp
from jax import lax
from jax.experimental import pallas as pl
from jax.experimental.pallas import tpu as pltpu
from jax.experimental.pallas import tpu_sc as plsc

BATCH = 4
SEQ = 2048
TOKENS = BATCH * SEQ
D = 1024
LN_EPS = 1e-3

NW = 32
TPW = TOKENS // NW
CHUNK = 32
N_CHUNKS = TPW // CHUNK

_sc_mesh = plsc.VectorSubcoreMesh(core_axis_name="c", subcore_axis_name="s")


@functools.partial(
    pl.kernel,
    mesh=_sc_mesh,
    out_type=jax.ShapeDtypeStruct((TOKENS, D), jnp.float32),
    scratch_types=[
        pltpu.VMEM((TPW,), jnp.int32),
        pltpu.VMEM((CHUNK, D), jnp.float32),
        pltpu.VMEM((CHUNK, D), jnp.float32),
        pltpu.SemaphoreType.DMA,
        pltpu.SemaphoreType.DMA,
    ],
)
def _sc_gather(idx_hbm, table_hbm, out_hbm, idx_v, rows_a, rows_b, sem_a, sem_b):
    wid = lax.axis_index("s") * 2 + lax.axis_index("c")
    base = wid * TPW
    pltpu.sync_copy(idx_hbm.at[pl.ds(base, TPW)], idx_v)

    pltpu.async_copy(table_hbm.at[idx_v.at[pl.ds(0, CHUNK)]], rows_a, sem_a)

    def body(i, _):
        buf = [rows_a, rows_b]
        sem = [sem_a, sem_b]
        for p in range(2):
            @pl.when(lax.rem(i, 2) == p)
            def _():
                cur, nxt = buf[p], buf[1 - p]
                csem, nsem = sem[p], sem[1 - p]
                @pl.when(i + 1 < N_CHUNKS)
                def _():
                    pltpu.async_copy(
                        table_hbm.at[idx_v.at[pl.ds((i + 1) * CHUNK, CHUNK)]],
                        nxt, nsem)
                pltpu.make_async_copy(
                    table_hbm.at[idx_v.at[pl.ds(i * CHUNK, CHUNK)]], cur, csem
                ).wait()
                pltpu.sync_copy(cur, out_hbm.at[pl.ds(base + i * CHUNK, CHUNK)])
        return 0

    lax.fori_loop(0, N_CHUNKS, body, 0)


TC_BLK = 512


def _tc_ln_body(gathered_ref, pos_ref, tsel_ref, type_emb_ref, gamma_ref,
                beta_ref, out_ref):
    x = gathered_ref[...]
    t0 = type_emb_ref[0:1, :]
    t1 = type_emb_ref[1:2, :]
    tsel = tsel_ref[...]
    x = x + pos_ref[...] + t0 + tsel * (t1 - t0)
    mean = jnp.mean(x, axis=-1, keepdims=True)
    xc = x - mean
    var = jnp.mean(xc * xc, axis=-1, keepdims=True)
    y = xc * lax.rsqrt(var + LN_EPS)
    out_ref[...] = y * gamma_ref[...] + beta_ref[...]


_BLOCKS_PER_BATCH = SEQ // TC_BLK


_tc_ln = pl.pallas_call(
    _tc_ln_body,
    grid=(_BLOCKS_PER_BATCH, BATCH),
    in_specs=[
        pl.BlockSpec((TC_BLK, D), lambda j, b: (b * _BLOCKS_PER_BATCH + j, 0)),
        pl.BlockSpec((TC_BLK, D), lambda j, b: (j, 0)),
        pl.BlockSpec((TC_BLK, 1), lambda j, b: (b * _BLOCKS_PER_BATCH + j, 0)),
        pl.BlockSpec((2, D), lambda j, b: (0, 0)),
        pl.BlockSpec((1, D), lambda j, b: (0, 0)),
        pl.BlockSpec((1, D), lambda j, b: (0, 0)),
    ],
    out_specs=pl.BlockSpec((TC_BLK, D), lambda j, b: (b * _BLOCKS_PER_BATCH + j, 0)),
    out_shape=jax.ShapeDtypeStruct((TOKENS, D), jnp.float32),
)


def kernel(input_word_ids, input_type_ids, word_emb, pos_emb, type_emb,
           ln_gamma, ln_beta):
    ids32 = input_word_ids.reshape(TOKENS).astype(jnp.int32)
    gathered = _sc_gather(ids32, word_emb)
    tsel = input_type_ids.reshape(TOKENS, 1).astype(jnp.float32)
    out = _tc_ln(gathered, pos_emb, tsel, type_emb,
                 ln_gamma.reshape(1, D), ln_beta.reshape(1, D))
    return out.reshape(BATCH, SEQ, D)
